# baseline (speedup 1.0000x reference)
import functools

import jax
import jax.numpy as jnp
from jax import lax
from jax.experimental import pallas as pl
from jax.experimental.pallas import tpu as pltpu
from jax.experimental.pallas import tpu_sc as plsc

NUM_EMB = 1000000
DIM = 64
BATCH = 16384
FIELDS = 26
B_TOTAL = BATCH * FIELDS
NW = 32
B_PER_W = B_TOTAL // NW
GATHER_W = 128
CHUNK = 512
ROWS_PER_CHUNK = CHUNK // GATHER_W
N_CHUNK = B_PER_W // CHUNK
IDX_ROWS = B_TOTAL // GATHER_W
ROWS_PER_W = IDX_ROWS // NW


def _make_gather():
    mesh = plsc.VectorSubcoreMesh(core_axis_name="c", subcore_axis_name="s")

    @functools.partial(
        pl.kernel,
        mesh=mesh,
        out_type=jax.ShapeDtypeStruct((B_TOTAL, DIM), jnp.float32),
        scratch_types=[
            pltpu.VMEM((ROWS_PER_CHUNK, GATHER_W), jnp.int32),
            pltpu.VMEM((CHUNK, DIM), jnp.float32),
            pltpu.SemaphoreType.DMA,
        ],
        compiler_params=pltpu.CompilerParams(use_tc_tiling_on_sc=False),
    )
    def gather_kernel(idx_hbm, table_hbm, out_hbm, idx_v, rows_v, sem):
        nc = 2
        wid = lax.axis_index("s") * nc + lax.axis_index("c")
        idx_row_base = wid * ROWS_PER_W
        out_base = wid * B_PER_W

        def body(g, carry):
            row_off = idx_row_base + g * ROWS_PER_CHUNK
            off = out_base + g * CHUNK
            pltpu.sync_copy(idx_hbm.at[pl.ds(row_off, ROWS_PER_CHUNK)], idx_v)
            copies = []
            for j in range(ROWS_PER_CHUNK):
                copies.append(
                    pltpu.async_copy(
                        table_hbm.at[idx_v.at[j]],
                        rows_v.at[pl.ds(j * GATHER_W, GATHER_W)],
                        sem,
                    )
                )
            for cp in copies:
                cp.wait()
            pltpu.sync_copy(rows_v, out_hbm.at[pl.ds(off, CHUNK)])
            return carry

        lax.fori_loop(0, N_CHUNK, body, 0)

    return gather_kernel


_gather = _make_gather()


def kernel(tokens_ids, weights):
    idx = tokens_ids.astype(jnp.int32).reshape(IDX_ROWS, GATHER_W)
    out = _gather(idx, weights)
    return out.reshape(BATCH, FIELDS, DIM)


# speedup vs baseline: 1.0303x; 1.0303x over previous
import functools

import jax
import jax.numpy as jnp
from jax import lax
from jax.experimental import pallas as pl
from jax.experimental.pallas import tpu as pltpu
from jax.experimental.pallas import tpu_sc as plsc

NUM_EMB = 1000000
DIM = 64
BATCH = 16384
FIELDS = 26
B_TOTAL = BATCH * FIELDS
NW = 32
B_PER_W = B_TOTAL // NW
GATHER_W = 128
CHUNK = 256
ROWS_PER_CHUNK = CHUNK // GATHER_W
N_CHUNK = B_PER_W // CHUNK
NBUF = 4
T_OUTER = N_CHUNK // NBUF
IDX_ROWS = B_TOTAL // GATHER_W
ROWS_PER_W = IDX_ROWS // NW


def _make_gather():
    mesh = plsc.VectorSubcoreMesh(core_axis_name="c", subcore_axis_name="s")

    @functools.partial(
        pl.kernel,
        mesh=mesh,
        out_type=jax.ShapeDtypeStruct((B_TOTAL, DIM), jnp.float32),
        scratch_types=[
            pltpu.VMEM((NBUF, ROWS_PER_CHUNK, GATHER_W), jnp.int32),
            pltpu.VMEM((NBUF, CHUNK, DIM), jnp.float32),
            pltpu.SemaphoreType.DMA,
            pltpu.SemaphoreType.DMA,
            pltpu.SemaphoreType.DMA,
            pltpu.SemaphoreType.DMA,
            pltpu.SemaphoreType.DMA,
        ],
        compiler_params=pltpu.CompilerParams(use_tc_tiling_on_sc=False),
    )
    def gather_kernel(idx_hbm, table_hbm, out_hbm, idx_v, rows_v,
                      gs0, gs1, gs2, gs3, osem):
        gsems = [gs0, gs1, gs2, gs3]
        nc = 2
        wid = lax.axis_index("s") * nc + lax.axis_index("c")
        idx_row_base = wid * ROWS_PER_W
        out_base = wid * B_PER_W

        def load_and_fire(g, b):
            row_off = idx_row_base + g * ROWS_PER_CHUNK
            pltpu.sync_copy(idx_hbm.at[pl.ds(row_off, ROWS_PER_CHUNK)],
                            idx_v.at[b])
            for j in range(ROWS_PER_CHUNK):
                pltpu.async_copy(
                    table_hbm.at[idx_v.at[b, j]],
                    rows_v.at[b, pl.ds(j * GATHER_W, GATHER_W)],
                    gsems[b],
                )

        def drain_gather(b):
            for j in range(ROWS_PER_CHUNK):
                pltpu.make_async_copy(
                    table_hbm.at[pl.ds(0, GATHER_W)],
                    rows_v.at[b, pl.ds(j * GATHER_W, GATHER_W)],
                    gsems[b],
                ).wait()

        for b in range(NBUF):
            load_and_fire(b, b)

        def body(t, carry):
            for b in range(NBUF):
                g = t * NBUF + b
                drain_gather(b)
                cp = pltpu.async_copy(
                    rows_v.at[b],
                    out_hbm.at[pl.ds(out_base + g * CHUNK, CHUNK)],
                    osem,
                )
                row_off = idx_row_base + (g + NBUF) * ROWS_PER_CHUNK
                pltpu.sync_copy(idx_hbm.at[pl.ds(row_off, ROWS_PER_CHUNK)],
                                idx_v.at[b])
                cp.wait()
                for j in range(ROWS_PER_CHUNK):
                    pltpu.async_copy(
                        table_hbm.at[idx_v.at[b, j]],
                        rows_v.at[b, pl.ds(j * GATHER_W, GATHER_W)],
                        gsems[b],
                    )
            return carry

        lax.fori_loop(0, T_OUTER - 1, body, 0)

        stores = []
        for b in range(NBUF):
            g = (T_OUTER - 1) * NBUF + b
            drain_gather(b)
            stores.append(
                pltpu.async_copy(
                    rows_v.at[b],
                    out_hbm.at[pl.ds(out_base + g * CHUNK, CHUNK)],
                    osem,
                )
            )
        for cp in stores:
            cp.wait()

    return gather_kernel


_gather = _make_gather()


def kernel(tokens_ids, weights):
    idx = tokens_ids.astype(jnp.int32).reshape(IDX_ROWS, GATHER_W)
    out = _gather(idx, weights)
    return out.reshape(BATCH, FIELDS, DIM)
